# baseline (speedup 1.0000x reference)
import functools

import jax
import jax.numpy as jnp
from jax import lax
from jax.experimental import pallas as pl
from jax.experimental.pallas import tpu as pltpu
from jax.experimental.pallas import tpu_sc as plsc

_TOPK = (1, 5)
_THR = 0.0
_G = 16
_R_SC = 512
_SC_CHUNK = 20000
_SC_UNROLL = 8


_SC_W = 4992
_SC_COLS = 99840


def _sc_count_ranks(pred, s1d, t32, row0, r_sc):
    info = plsc.get_sparse_core_info()
    nw = info.num_cores * info.num_subcores
    rpw = r_sc // nw
    assert r_sc % nw == 0 and rpw % 8 == 0 and row0 % 8 == 0
    rpad = ((rpw + 15) // 16) * 16
    w = _SC_W
    nch = _SC_COLS // w
    assert _SC_COLS % w == 0 and (w // 16) % _SC_UNROLL == 0
    steps = w // (16 * _SC_UNROLL)

    mesh = plsc.VectorSubcoreMesh(core_axis_name="c", subcore_axis_name="s")

    @functools.partial(
        pl.kernel,
        mesh=mesh,
        out_type=jax.ShapeDtypeStruct((r_sc, 16), jnp.int32),
        scratch_types=[
            pltpu.VMEM((rpad,), jnp.float32),
            pltpu.VMEM((rpad,), jnp.int32),
            pltpu.VMEM((rpad, 16), jnp.int32),
            pltpu.VMEM((8, w), jnp.float32),
            pltpu.VMEM((8, w), jnp.float32),
            pltpu.SemaphoreType.DMA,
            pltpu.SemaphoreType.DMA,
        ],
    )
    def count_kernel(pred_hbm, s_hbm, t_hbm, out_hbm,
                     s_v, t_v, rank_v, buf0, buf1, sem0, sem1):
        wid = lax.axis_index("s") * info.num_cores + lax.axis_index("c")
        base = row0 + wid * rpw
        pltpu.sync_copy(s_hbm.at[pl.ds(base, rpw)], s_v.at[pl.ds(0, rpw)])
        pltpu.sync_copy(t_hbm.at[pl.ds(base, rpw)], t_v.at[pl.ds(0, rpw)])
        bufs = (buf0, buf1)
        sems = (sem0, sem1)
        lane_iota = lax.iota(jnp.int32, 16)

        for b in range(rpw // 8):

            def chunk_start(ch, slot, band=b):
                off = pl.multiple_of(ch * w, w)
                return pltpu.async_copy(
                    pred_hbm.at[pl.ds(base + band * 8, 8), pl.ds(off, w)],
                    bufs[slot], sems[slot])

            spls = []
            for lr in range(8):
                r = b * 8 + lr
                g = r // 16
                s16 = s_v[pl.ds(g * 16, 16)]
                t16 = t_v[pl.ds(g * 16, 16)]
                spls.append((lax.broadcast(s16[r % 16], (16,)),
                             lax.broadcast(t16[r % 16], (16,))))

            def process(buf, ch, accs):
                def step(j, carry):
                    aa = list(carry[:8])
                    cc0 = carry[8]
                    for u in range(_SC_UNROLL):
                        cc = cc0 + u * 16
                        off = j * (16 * _SC_UNROLL) + u * 16
                        for lr in range(8):
                            s_spl, t_spl = spls[lr]
                            v = buf[lr, pl.ds(off, 16)]
                            gt = v > s_spl
                            tie = (v == s_spl) & (cc < t_spl)
                            aa[lr] = (aa[lr] + jnp.where(gt, 1, 0)
                                      + jnp.where(tie, 1, 0))
                    return tuple(aa) + (cc0 + 16 * _SC_UNROLL,)

                col0 = ch * w + lane_iota
                res = lax.fori_loop(0, steps, step, tuple(accs) + (col0,))
                return list(res[:8])

            def chunk_wait(slot, band=b):
                pltpu.make_async_copy(
                    pred_hbm.at[pl.ds(base + band * 8, 8), pl.ds(0, w)],
                    bufs[slot], sems[slot]).wait()

            accs = [jnp.zeros((16,), jnp.int32) for _ in range(8)]
            chunk_start(0, 0)

            def pair(p, carry):
                accs = list(carry)
                ch0 = p * 2
                chunk_start(jnp.minimum(ch0 + 1, nch - 1), 1)
                chunk_wait(0)
                accs = process(bufs[0], ch0, accs)
                chunk_start(jnp.minimum(ch0 + 2, nch - 1), 0)
                chunk_wait(1)
                accs = process(bufs[1], ch0 + 1, accs)
                return tuple(accs)

            accs = list(lax.fori_loop(0, nch // 2, pair, tuple(accs)))
            chunk_wait(0)
            for lr in range(8):
                rank_v[b * 8 + lr, :] = accs[lr]
        pltpu.sync_copy(rank_v.at[pl.ds(0, rpw)],
                        out_hbm.at[pl.ds(wid * rpw, rpw)])

    return count_kernel(pred, s1d, t32)


def _tc_finalize(rank_tc, rank_sc, tail_cnt, s2d, r_tc, num_rows):
    def body(rtc_ref, rsc_ref, tail_ref, s_ref, out_ref):
        rtc = rtc_ref[...]
        rsc = (jnp.sum(rsc_ref[...], axis=1, keepdims=True)
               + tail_ref[...])
        s = s_ref[...]
        ok_tc = s[0:r_tc, :] > _THR
        ok_sc = s[r_tc:, :] > _THR
        t1 = (jnp.sum(((rtc < _TOPK[0]) & ok_tc).astype(jnp.float32))
              + jnp.sum(((rsc < _TOPK[0]) & ok_sc).astype(jnp.float32)))
        t5 = (jnp.sum(((rtc < _TOPK[1]) & ok_tc).astype(jnp.float32))
              + jnp.sum(((rsc < _TOPK[1]) & ok_sc).astype(jnp.float32)))
        lanes = lax.broadcasted_iota(jnp.int32, (1, 2), 1)
        out_ref[...] = jnp.where(lanes == 0, t1, t5) * (100.0 / num_rows)

    return pl.pallas_call(
        body,
        out_shape=jax.ShapeDtypeStruct((1, 2), jnp.float32),
    )(rank_tc, rank_sc, tail_cnt, s2d)


def _tc_gather_scores(pred, t32, num_rows, num_cols):
    assert num_rows % _G == 0

    def body(t_ref, *refs):
        i = pl.program_id(0)
        out_ref = refs[_G]
        rowg = lax.broadcasted_iota(jnp.int32, (_G, 1), 0)
        row8 = lax.broadcasted_iota(jnp.int32, (8, 128), 0)
        lane = lax.broadcasted_iota(jnp.int32, (8, 128), 1)
        acc = jnp.zeros((_G, 1), jnp.float32)
        for k in range(_G):
            r = i * _G + k
            off = t_ref[r] & 127
            v = refs[k][...]
            picked = jnp.where((row8 == (r & 7)) & (lane == off), v, 0.0)
            acc = acc + jnp.where(rowg == k, jnp.sum(picked), 0.0)
        out_ref[...] = acc

    def mk_spec(k):
        return pl.BlockSpec(
            (8, 128),
            lambda i, tref, k=k: ((i * _G + k) // 8, tref[i * _G + k] // 128))

    return pl.pallas_call(
        body,
        grid_spec=pltpu.PrefetchScalarGridSpec(
            num_scalar_prefetch=1,
            grid=(num_rows // _G,),
            in_specs=[mk_spec(k) for k in range(_G)],
            out_specs=pl.BlockSpec((_G, 1), lambda i, tref: (i, 0)),
        ),
        out_shape=jax.ShapeDtypeStruct((num_rows, 1), jnp.float32),
        compiler_params=pltpu.CompilerParams(
            dimension_semantics=("arbitrary",)),
    )(t32, *([pred] * _G))


def _tc_rank_scan(pred, t2d, s2d, t_sc2d, s_sc2d, r_tc, r_sc, num_cols, cb):
    nb = (num_cols + cb - 1) // cb
    col0l = (nb - 1) * cb
    assert r_tc == r_sc and col0l <= _SC_COLS

    def body(pred_ref, tailp_ref, t_ref, s_ref, tsc_ref, ssc_ref,
             out_ref, tail_ref, acc_ref):
        c = pl.program_id(0)

        @pl.when(c == 0)
        def _init():
            acc_ref[...] = jnp.zeros_like(acc_ref)

        v = pred_ref[...]
        s = s_ref[...]
        t = t_ref[...]
        col0 = c * cb
        rel = lax.broadcasted_iota(jnp.int32, (r_tc, cb), 1)
        eqb = (v == s) & (rel < (t - col0))

        @pl.when(c < nb - 1)
        def _mid():
            cnt = ((v > s) | eqb).astype(jnp.int32)
            part = cnt[:, 0:128]
            for k in range(1, cb // 128):
                part = part + cnt[:, k * 128:(k + 1) * 128]
            acc_ref[...] += part

        @pl.when(c == nb - 1)
        def _last():
            gt = (v > s) & (rel < (num_cols - col0))
            cnt = (gt | eqb).astype(jnp.int32)
            part = cnt[:, 0:128]
            for k in range(1, cb // 128):
                part = part + cnt[:, k * 128:(k + 1) * 128]
            acc_ref[...] += part
            out_ref[...] = jnp.sum(acc_ref[...], axis=1, keepdims=True)

            vt = tailp_ref[...]
            ssc = ssc_ref[...]
            tsc = tsc_ref[...]
            gt2 = (vt > ssc) & (rel < (num_cols - col0l))
            eqb2 = (vt == ssc) & (rel < (tsc - col0l))
            tcnt = ((gt2 | eqb2) &
                    (rel >= (_SC_COLS - col0l))).astype(jnp.int32)
            tail_ref[...] = jnp.sum(tcnt, axis=1, keepdims=True)

    return pl.pallas_call(
        body,
        grid=(nb,),
        in_specs=[
            pl.BlockSpec((r_tc, cb), lambda c: (0, c)),
            pl.BlockSpec((r_sc, cb), lambda c: (1, nb - 1)),
            pl.BlockSpec((r_tc, 1), lambda c: (0, 0)),
            pl.BlockSpec((r_tc, 1), lambda c: (0, 0)),
            pl.BlockSpec((r_sc, 1), lambda c: (0, 0)),
            pl.BlockSpec((r_sc, 1), lambda c: (0, 0)),
        ],
        out_specs=[
            pl.BlockSpec((r_tc, 1), lambda c: (0, 0)),
            pl.BlockSpec((r_sc, 1), lambda c: (0, 0)),
        ],
        out_shape=[jax.ShapeDtypeStruct((r_tc, 1), jnp.int32),
                   jax.ShapeDtypeStruct((r_sc, 1), jnp.int32)],
        scratch_shapes=[pltpu.VMEM((r_tc, 128), jnp.int32)],
        compiler_params=pltpu.CompilerParams(
            dimension_semantics=("arbitrary",)),
    )(pred, pred, t2d, s2d, t_sc2d, s_sc2d)


def kernel(pred, target):
    num_rows, num_cols = pred.shape
    r_sc = _R_SC
    r_tc = num_rows - r_sc
    t32 = target.astype(jnp.int32)
    t2d = t32.reshape(num_rows, 1)
    s2d = _tc_gather_scores(pred, t32, num_rows, num_cols)
    rank_sc = _sc_count_ranks(pred, s2d.reshape(-1), t32, r_tc, r_sc)
    rank_tc, tail_cnt = _tc_rank_scan(pred, t2d[:r_tc], s2d[:r_tc],
                                      t2d[r_tc:], s2d[r_tc:],
                                      r_tc, r_sc, num_cols, cb=2048)
    return _tc_finalize(rank_tc, rank_sc, tail_cnt, s2d,
                        r_tc, num_rows).reshape(2)


# speedup vs baseline: 3.6111x; 3.6111x over previous
import jax
import jax.numpy as jnp
from jax import lax
from jax.experimental import pallas as pl
from jax.experimental.pallas import tpu as pltpu

_TOPK = (1, 5)
_THR = 0.0
_G = 16


def _tc_gather_scores(pred, t32, num_rows, num_cols):
    assert num_rows % _G == 0

    def body(t_ref, *refs):
        i = pl.program_id(0)
        out_ref = refs[_G]
        rowg = lax.broadcasted_iota(jnp.int32, (_G, 1), 0)
        row8 = lax.broadcasted_iota(jnp.int32, (8, 128), 0)
        lane = lax.broadcasted_iota(jnp.int32, (8, 128), 1)
        acc = jnp.zeros((_G, 1), jnp.float32)
        for k in range(_G):
            r = i * _G + k
            off = t_ref[r] & 127
            v = refs[k][...]
            picked = jnp.where((row8 == (r & 7)) & (lane == off), v, 0.0)
            acc = acc + jnp.where(rowg == k, jnp.sum(picked), 0.0)
        out_ref[...] = acc

    def mk_spec(k):
        return pl.BlockSpec(
            (8, 128),
            lambda i, tref, k=k: ((i * _G + k) // 8, tref[i * _G + k] // 128))

    return pl.pallas_call(
        body,
        grid_spec=pltpu.PrefetchScalarGridSpec(
            num_scalar_prefetch=1,
            grid=(num_rows // _G,),
            in_specs=[mk_spec(k) for k in range(_G)],
            out_specs=pl.BlockSpec((_G, 1), lambda i, tref: (i, 0)),
        ),
        out_shape=jax.ShapeDtypeStruct((num_rows, 1), jnp.float32),
        compiler_params=pltpu.CompilerParams(
            dimension_semantics=("arbitrary",)),
    )(t32, *([pred] * _G))


def _tc_rank_scan(pred, t2d, s2d, num_rows, num_cols, cb):
    nb = (num_cols + cb - 1) // cb

    def body(pred_ref, t_ref, s_ref, out_ref, acc_ref):
        c = pl.program_id(0)

        @pl.when(c == 0)
        def _init():
            acc_ref[...] = jnp.zeros_like(acc_ref)

        v = pred_ref[...]
        s = s_ref[...]
        t = t_ref[...]
        col0 = c * cb
        rel = lax.broadcasted_iota(jnp.int32, (num_rows, cb), 1)
        eqb = (v == s) & (rel < (t - col0))

        @pl.when(c < nb - 1)
        def _mid():
            cnt = ((v > s) | eqb).astype(jnp.int32)
            part = cnt[:, 0:128]
            for k in range(1, cb // 128):
                part = part + cnt[:, k * 128:(k + 1) * 128]
            acc_ref[...] += part

        @pl.when(c == nb - 1)
        def _last():
            gt = (v > s) & (rel < (num_cols - col0))
            cnt = (gt | eqb).astype(jnp.int32)
            part = cnt[:, 0:128]
            for k in range(1, cb // 128):
                part = part + cnt[:, k * 128:(k + 1) * 128]
            acc_ref[...] += part

            rank = jnp.sum(acc_ref[...], axis=1, keepdims=True)
            ok = s > _THR
            t1 = jnp.sum(((rank < _TOPK[0]) & ok).astype(jnp.float32))
            t5 = jnp.sum(((rank < _TOPK[1]) & ok).astype(jnp.float32))
            lanes = lax.broadcasted_iota(jnp.int32, (1, 2), 1)
            out_ref[...] = jnp.where(lanes == 0, t1, t5) * (100.0 / num_rows)

    return pl.pallas_call(
        body,
        grid=(nb,),
        in_specs=[
            pl.BlockSpec((num_rows, cb), lambda c: (0, c)),
            pl.BlockSpec((num_rows, 1), lambda c: (0, 0)),
            pl.BlockSpec((num_rows, 1), lambda c: (0, 0)),
        ],
        out_specs=pl.BlockSpec((1, 2), lambda c: (0, 0)),
        out_shape=jax.ShapeDtypeStruct((1, 2), jnp.float32),
        scratch_shapes=[pltpu.VMEM((num_rows, 128), jnp.int32)],
        compiler_params=pltpu.CompilerParams(
            dimension_semantics=("arbitrary",)),
    )(pred, t2d, s2d)


def kernel(pred, target):
    num_rows, num_cols = pred.shape
    t32 = target.astype(jnp.int32)
    s2d = _tc_gather_scores(pred, t32, num_rows, num_cols)
    out = _tc_rank_scan(pred, t32.reshape(num_rows, 1), s2d,
                        num_rows, num_cols, cb=2048)
    return out.reshape(2)
